# baseline (speedup 1.0000x reference)
import jax
import jax.numpy as jnp
from jax.experimental import pallas as pl

B, C, H, W = 8, 96, 224, 224
HB = 56


def _body(x_ref, o_ref):
    v = x_ref[0]
    m = jnp.max(v, axis=0, keepdims=True)
    e = jnp.exp(v - m)
    s = jnp.sum(e, axis=0, keepdims=True)
    base = m + jnp.log(s)
    for i in range(HB):
        o_ref[0, :, i, :] = (v[:, i, :] - base[0, i, :]).T


def kernel(x):
    out = pl.pallas_call(
        _body,
        grid=(B, H // HB),
        in_specs=[pl.BlockSpec((1, C, HB, W), lambda b, h: (b, 0, h, 0))],
        out_specs=pl.BlockSpec((1, W, HB, C), lambda b, h: (b, 0, h, 0)),
        out_shape=jax.ShapeDtypeStruct((B, W, H, C), x.dtype),
    )(x)
    return out.reshape(-1, C)
